# baseline (speedup 1.0000x reference)
import functools

import jax
import jax.numpy as jnp
from jax import lax
from jax.experimental import pallas as pl
from jax.experimental.pallas import tpu as pltpu
from jax.experimental.pallas import tpu_sc as plsc

_B, _D, _A, _S = 16384, 128, 64, 64
_BT = 1024
_NCOL = 192
_NPAD = _NCOL - (_A + _S + 1)

_NC, _NS = 2, 16
_NW = _NC * _NS
_STD_TOTAL = _B * _A
_PER_W = _STD_TOTAL // _NW
_CHUNK = 4096
_NDMA = _PER_W // _CHUNK


def _tc_body(state_ref, w_ref, b_ref, mean_ref, value_ref, onehot_ref):
    x = state_ref[...]
    w = w_ref[...]
    y = jax.lax.dot_general(x, w, (((1,), (0,)), ((), ())),
                            preferred_element_type=jnp.float32)
    y = y + b_ref[...]
    mean_ref[...] = y[:, :_A]
    logits = y[:, _A:_A + _S]
    idx = jnp.argmax(logits, axis=-1)
    iota = jax.lax.broadcasted_iota(jnp.int32, (_BT, _S), 1)
    onehot_ref[...] = (iota == idx[:, None]).astype(jnp.float32)
    value_ref[...] = y[:, _A + _S:_A + _S + 1]


def _sc_std_body(out_hbm, zbuf, sem):
    wid = lax.axis_index("s") * _NC + lax.axis_index("c")
    base = wid * _PER_W

    def _zero(i, carry):
        zbuf[pl.ds(i * 16, 16)] = jnp.zeros((16,), jnp.float32)
        return carry

    lax.fori_loop(0, _CHUNK // 16, _zero, 0)
    copies = [
        pltpu.async_copy(zbuf, out_hbm.at[pl.ds(base + k * _CHUNK, _CHUNK)], sem)
        for k in range(_NDMA)
    ]
    for c in copies:
        c.wait()


@functools.cache
def _make_sc_std_zeros():
    return pl.kernel(
        _sc_std_body,
        out_type=jax.ShapeDtypeStruct((_STD_TOTAL,), jnp.float32),
        mesh=plsc.VectorSubcoreMesh(core_axis_name="c", subcore_axis_name="s",
                                    num_cores=_NC, num_subcores=_NS),
        scratch_types=[
            pltpu.VMEM((_CHUNK,), jnp.float32),
            pltpu.SemaphoreType.DMA,
        ],
    )


def kernel(state, W_skill, b_skill, W_action, b_action, W_value, b_value):
    w_cat = jnp.concatenate(
        [W_action, W_skill, W_value,
         jnp.zeros((_NPAD, _D), jnp.float32)], axis=0).T
    b_cat = jnp.concatenate(
        [b_action, b_skill, b_value,
         jnp.zeros((_NPAD,), jnp.float32)]).reshape(1, _NCOL)

    mean, value, one_hot = pl.pallas_call(
        _tc_body,
        grid=(_B // _BT,),
        in_specs=[
            pl.BlockSpec((_BT, _D), lambda i: (i, 0)),
            pl.BlockSpec((_D, _NCOL), lambda i: (0, 0)),
            pl.BlockSpec((1, _NCOL), lambda i: (0, 0)),
        ],
        out_specs=[
            pl.BlockSpec((_BT, _A), lambda i: (i, 0)),
            pl.BlockSpec((_BT, 1), lambda i: (i, 0)),
            pl.BlockSpec((_BT, _S), lambda i: (i, 0)),
        ],
        out_shape=[
            jax.ShapeDtypeStruct((_B, _A), jnp.float32),
            jax.ShapeDtypeStruct((_B, 1), jnp.float32),
            jax.ShapeDtypeStruct((_B, _S), jnp.float32),
        ],
        compiler_params=pltpu.CompilerParams(
            dimension_semantics=("arbitrary",),
        ),
    )(state, w_cat, b_cat)

    std = _make_sc_std_zeros()().reshape(_B, _A)
    return (mean, std, value.reshape(_B), one_hot)


# speedup vs baseline: 1.1945x; 1.1945x over previous
import functools

import jax
import jax.numpy as jnp
from jax import lax
from jax.experimental import pallas as pl
from jax.experimental.pallas import tpu as pltpu
from jax.experimental.pallas import tpu_sc as plsc

_B, _D, _A, _S = 16384, 128, 64, 64
_BT = 1024
_NCOL = 192
_NPAD = _NCOL - (_A + _S + 1)

_NC, _NS = 2, 16
_NW = _NC * _NS
_ROWS_W = _B // _NW
_CHUNK_ROWS = 64
_NDMA = _ROWS_W // _CHUNK_ROWS


def _tc_body(state_ref, w_ref, b_ref, mean_ref, value_ref, onehot_ref):
    x = state_ref[...]
    w = w_ref[...]
    y = jax.lax.dot_general(x, w, (((1,), (0,)), ((), ())),
                            preferred_element_type=jnp.float32)
    y = y + b_ref[...]
    mean_ref[...] = y[:, :_A]
    logits = y[:, _A:_A + _S]
    idx = jnp.argmax(logits, axis=-1)
    iota = jax.lax.broadcasted_iota(jnp.int32, (_BT, _S), 1)
    onehot_ref[...] = (iota == idx[:, None]).astype(jnp.float32)
    value_ref[...] = y[:, _A + _S]


def _sc_std_body(out_hbm, zbuf, sem):
    wid = lax.axis_index("s") * _NC + lax.axis_index("c")
    row0 = wid * _ROWS_W

    def _zero(i, carry):
        zbuf[i, pl.ds(0, 16)] = jnp.zeros((16,), jnp.float32)
        zbuf[i, pl.ds(16, 16)] = jnp.zeros((16,), jnp.float32)
        zbuf[i, pl.ds(32, 16)] = jnp.zeros((16,), jnp.float32)
        zbuf[i, pl.ds(48, 16)] = jnp.zeros((16,), jnp.float32)
        return carry

    lax.fori_loop(0, _CHUNK_ROWS, _zero, 0)
    copies = [
        pltpu.async_copy(
            zbuf, out_hbm.at[pl.ds(row0 + k * _CHUNK_ROWS, _CHUNK_ROWS), :], sem)
        for k in range(_NDMA)
    ]
    for c in copies:
        c.wait()


@functools.cache
def _make_sc_std_zeros():
    return pl.kernel(
        _sc_std_body,
        out_type=jax.ShapeDtypeStruct((_B, _A), jnp.float32),
        mesh=plsc.VectorSubcoreMesh(core_axis_name="c", subcore_axis_name="s",
                                    num_cores=_NC, num_subcores=_NS),
        scratch_types=[
            pltpu.VMEM((_CHUNK_ROWS, _A), jnp.float32),
            pltpu.SemaphoreType.DMA,
        ],
    )


def kernel(state, W_skill, b_skill, W_action, b_action, W_value, b_value):
    w_cat = jnp.concatenate(
        [W_action, W_skill, W_value,
         jnp.zeros((_NPAD, _D), jnp.float32)], axis=0).T
    b_cat = jnp.concatenate(
        [b_action, b_skill, b_value,
         jnp.zeros((_NPAD,), jnp.float32)]).reshape(1, _NCOL)

    mean, value, one_hot = pl.pallas_call(
        _tc_body,
        grid=(_B // _BT,),
        in_specs=[
            pl.BlockSpec((_BT, _D), lambda i: (i, 0)),
            pl.BlockSpec((_D, _NCOL), lambda i: (0, 0)),
            pl.BlockSpec((1, _NCOL), lambda i: (0, 0)),
        ],
        out_specs=[
            pl.BlockSpec((_BT, _A), lambda i: (i, 0)),
            pl.BlockSpec((_BT,), lambda i: (i,)),
            pl.BlockSpec((_BT, _S), lambda i: (i, 0)),
        ],
        out_shape=[
            jax.ShapeDtypeStruct((_B, _A), jnp.float32),
            jax.ShapeDtypeStruct((_B,), jnp.float32),
            jax.ShapeDtypeStruct((_B, _S), jnp.float32),
        ],
        compiler_params=pltpu.CompilerParams(
            dimension_semantics=("arbitrary",),
        ),
    )(state, w_cat, b_cat)

    std = _make_sc_std_zeros()()
    return (mean, std, value, one_hot)


# speedup vs baseline: 2.2168x; 1.8558x over previous
import functools

import jax
import jax.numpy as jnp
from jax import lax
from jax.experimental import pallas as pl
from jax.experimental.pallas import tpu as pltpu
from jax.experimental.pallas import tpu_sc as plsc

_B, _D, _A, _S = 16384, 128, 64, 64
_BT = 1024
_NROW = 192
_NPAD = _NROW - (_A + _S + 1)

_NC, _NS = 2, 16
_NW = _NC * _NS
_ROWS_W = _A // _NW * 2
_CHUNK = 4096
_NCH = _B // _CHUNK


def _tc_body(state_ref, w_ref, b_ref, mean_ref, value_ref, onehot_ref):
    x = state_ref[...]
    w = w_ref[...]
    y = jax.lax.dot_general(w, x, (((1,), (1,)), ((), ())),
                            preferred_element_type=jnp.float32)
    y = y + b_ref[...]
    mean_ref[...] = y[:_A, :]
    logits = y[_A:_A + _S, :]
    m = jnp.max(logits, axis=0, keepdims=True)
    iota = jax.lax.broadcasted_iota(jnp.int32, (_S, _BT), 0)
    first = jnp.min(jnp.where(logits == m, iota, _S), axis=0, keepdims=True)
    onehot_ref[...] = (iota == first).astype(jnp.float32)
    value_ref[...] = y[_A + _S, :]


def _sc_std_body(out_hbm, zbuf, sem):
    wid = lax.axis_index("s") * _NC + lax.axis_index("c")
    row0 = wid * 2

    def _zero(i, carry):
        zbuf[0, pl.ds(i * 16, 16)] = jnp.zeros((16,), jnp.float32)
        return carry

    lax.fori_loop(0, _CHUNK // 16, _zero, 0)
    copies = [
        pltpu.async_copy(
            zbuf,
            out_hbm.at[pl.ds(row0 + k // _NCH, 1),
                       pl.ds((k % _NCH) * _CHUNK, _CHUNK)],
            sem)
        for k in range(2 * _NCH)
    ]
    for c in copies:
        c.wait()


@functools.cache
def _make_sc_std_zeros():
    return pl.kernel(
        _sc_std_body,
        out_type=jax.ShapeDtypeStruct((_A, _B), jnp.float32),
        mesh=plsc.VectorSubcoreMesh(core_axis_name="c", subcore_axis_name="s",
                                    num_cores=_NC, num_subcores=_NS),
        scratch_types=[
            pltpu.VMEM((1, _CHUNK), jnp.float32),
            pltpu.SemaphoreType.DMA,
        ],
    )


def kernel(state, W_skill, b_skill, W_action, b_action, W_value, b_value):
    w_rows = jnp.concatenate(
        [W_action, W_skill, W_value, jnp.zeros((_NPAD, _D), jnp.float32)],
        axis=0)
    b_col = jnp.concatenate(
        [b_action, b_skill, b_value,
         jnp.zeros((_NPAD,), jnp.float32)]).reshape(_NROW, 1)

    mean_t, value, onehot_t = pl.pallas_call(
        _tc_body,
        grid=(_B // _BT,),
        in_specs=[
            pl.BlockSpec((_BT, _D), lambda i: (i, 0)),
            pl.BlockSpec((_NROW, _D), lambda i: (0, 0)),
            pl.BlockSpec((_NROW, 1), lambda i: (0, 0)),
        ],
        out_specs=[
            pl.BlockSpec((_A, _BT), lambda i: (0, i)),
            pl.BlockSpec((_BT,), lambda i: (i,)),
            pl.BlockSpec((_S, _BT), lambda i: (0, i)),
        ],
        out_shape=[
            jax.ShapeDtypeStruct((_A, _B), jnp.float32),
            jax.ShapeDtypeStruct((_B,), jnp.float32),
            jax.ShapeDtypeStruct((_S, _B), jnp.float32),
        ],
        compiler_params=pltpu.CompilerParams(
            dimension_semantics=("arbitrary",),
        ),
    )(state, w_rows, b_col)

    std_t = _make_sc_std_zeros()()
    return (mean_t.T, std_t.T, value, onehot_t.T)


# speedup vs baseline: 3.6527x; 1.6478x over previous
import jax
import jax.numpy as jnp
from jax.experimental import pallas as pl
from jax.experimental.pallas import tpu as pltpu

_B, _D, _A, _S = 16384, 128, 64, 64
_BT = 1024
_NROW = 192
_NPAD = _NROW - (_A + _S + 1)


def _tc_body(state_ref, w_ref, b_ref, mean_ref, std_ref, value_ref, onehot_ref):
    x = state_ref[...]
    w = w_ref[...]
    y = jax.lax.dot_general(w, x, (((1,), (1,)), ((), ())),
                            preferred_element_type=jnp.float32)
    y = y + b_ref[...]
    mean_ref[...] = y[:_A, :]
    std_ref[...] = jnp.zeros((_A, _BT), jnp.float32)
    logits = y[_A:_A + _S, :]
    m = jnp.max(logits, axis=0, keepdims=True)
    iota = jax.lax.broadcasted_iota(jnp.int32, (_S, _BT), 0)
    first = jnp.min(jnp.where(logits == m, iota, _S), axis=0, keepdims=True)
    onehot_ref[...] = (iota == first).astype(jnp.float32)
    value_ref[...] = y[_A + _S, :]


def kernel(state, W_skill, b_skill, W_action, b_action, W_value, b_value):
    w_rows = jnp.concatenate(
        [W_action, W_skill, W_value, jnp.zeros((_NPAD, _D), jnp.float32)],
        axis=0)
    b_col = jnp.concatenate(
        [b_action, b_skill, b_value,
         jnp.zeros((_NPAD,), jnp.float32)]).reshape(_NROW, 1)

    mean_t, std_t, value, onehot_t = pl.pallas_call(
        _tc_body,
        grid=(_B // _BT,),
        in_specs=[
            pl.BlockSpec((_BT, _D), lambda i: (i, 0)),
            pl.BlockSpec((_NROW, _D), lambda i: (0, 0)),
            pl.BlockSpec((_NROW, 1), lambda i: (0, 0)),
        ],
        out_specs=[
            pl.BlockSpec((_A, _BT), lambda i: (0, i)),
            pl.BlockSpec((_A, _BT), lambda i: (0, i)),
            pl.BlockSpec((_BT,), lambda i: (i,)),
            pl.BlockSpec((_S, _BT), lambda i: (0, i)),
        ],
        out_shape=[
            jax.ShapeDtypeStruct((_A, _B), jnp.float32),
            jax.ShapeDtypeStruct((_A, _B), jnp.float32),
            jax.ShapeDtypeStruct((_B,), jnp.float32),
            jax.ShapeDtypeStruct((_S, _B), jnp.float32),
        ],
        compiler_params=pltpu.CompilerParams(
            dimension_semantics=("arbitrary",),
        ),
    )(state, w_rows, b_col)

    return (mean_t.T, std_t.T, value, onehot_t.T)


# speedup vs baseline: 4.7579x; 1.3026x over previous
import jax
import jax.numpy as jnp
from jax.experimental import pallas as pl
from jax.experimental.pallas import tpu as pltpu

_B, _D, _A, _S = 16384, 128, 64, 64
_BT = 2048
_NROW = 192
_NPAD = _NROW - (_A + _S + 1)


def _tc_body(state_ref, w_ref, b_ref, mean_ref, std_ref, value_ref, onehot_ref):
    x = state_ref[...]
    w = w_ref[...]
    y = jax.lax.dot_general(w, x, (((1,), (1,)), ((), ())),
                            preferred_element_type=jnp.float32)
    y = y + b_ref[...]
    mean_ref[...] = y[:_A, :]
    std_ref[...] = jnp.zeros((_A, _BT), jnp.float32)
    logits = y[_A:_A + _S, :]
    m = jnp.max(logits, axis=0, keepdims=True)
    iota = jax.lax.broadcasted_iota(jnp.int32, (_S, _BT), 0)
    first = jnp.min(jnp.where(logits == m, iota, _S), axis=0, keepdims=True)
    onehot_ref[...] = (iota == first).astype(jnp.float32)
    value_ref[...] = y[_A + _S, :]


def kernel(state, W_skill, b_skill, W_action, b_action, W_value, b_value):
    w_rows = jnp.concatenate(
        [W_action, W_skill, W_value, jnp.zeros((_NPAD, _D), jnp.float32)],
        axis=0)
    b_col = jnp.concatenate(
        [b_action, b_skill, b_value,
         jnp.zeros((_NPAD,), jnp.float32)]).reshape(_NROW, 1)

    mean_t, std_t, value, onehot_t = pl.pallas_call(
        _tc_body,
        grid=(_B // _BT,),
        in_specs=[
            pl.BlockSpec((_BT, _D), lambda i: (i, 0)),
            pl.BlockSpec((_NROW, _D), lambda i: (0, 0)),
            pl.BlockSpec((_NROW, 1), lambda i: (0, 0)),
        ],
        out_specs=[
            pl.BlockSpec((_A, _BT), lambda i: (0, i)),
            pl.BlockSpec((_A, _BT), lambda i: (0, i)),
            pl.BlockSpec((_BT,), lambda i: (i,)),
            pl.BlockSpec((_S, _BT), lambda i: (0, i)),
        ],
        out_shape=[
            jax.ShapeDtypeStruct((_A, _B), jnp.float32),
            jax.ShapeDtypeStruct((_A, _B), jnp.float32),
            jax.ShapeDtypeStruct((_B,), jnp.float32),
            jax.ShapeDtypeStruct((_S, _B), jnp.float32),
        ],
        compiler_params=pltpu.CompilerParams(
            dimension_semantics=("arbitrary",),
        ),
    )(state, w_rows, b_col)

    return (mean_t.T, std_t.T, value, onehot_t.T)


# speedup vs baseline: 5.4824x; 1.1523x over previous
import jax
import jax.numpy as jnp
from jax.experimental import pallas as pl
from jax.experimental.pallas import tpu as pltpu

_B, _D, _A, _S = 16384, 128, 64, 64
_BT = 4096
_NROW = 192
_NPAD = _NROW - (_A + _S + 1)


def _tc_body(state_ref, w_ref, b_ref, mean_ref, std_ref, value_ref, onehot_ref):
    x = state_ref[...]
    w = w_ref[...]
    y = jax.lax.dot_general(w, x, (((1,), (1,)), ((), ())),
                            preferred_element_type=jnp.float32)
    y = y + b_ref[...]
    mean_ref[...] = y[:_A, :]
    std_ref[...] = jnp.zeros((_A, _BT), jnp.float32)
    logits = y[_A:_A + _S, :]
    m = jnp.max(logits, axis=0, keepdims=True)
    iota = jax.lax.broadcasted_iota(jnp.int32, (_S, _BT), 0)
    first = jnp.min(jnp.where(logits == m, iota, _S), axis=0, keepdims=True)
    onehot_ref[...] = (iota == first).astype(jnp.float32)
    value_ref[...] = y[_A + _S, :]


def kernel(state, W_skill, b_skill, W_action, b_action, W_value, b_value):
    w_rows = jnp.concatenate(
        [W_action, W_skill, W_value, jnp.zeros((_NPAD, _D), jnp.float32)],
        axis=0)
    b_col = jnp.concatenate(
        [b_action, b_skill, b_value,
         jnp.zeros((_NPAD,), jnp.float32)]).reshape(_NROW, 1)

    mean_t, std_t, value, onehot_t = pl.pallas_call(
        _tc_body,
        grid=(_B // _BT,),
        in_specs=[
            pl.BlockSpec((_BT, _D), lambda i: (i, 0)),
            pl.BlockSpec((_NROW, _D), lambda i: (0, 0)),
            pl.BlockSpec((_NROW, 1), lambda i: (0, 0)),
        ],
        out_specs=[
            pl.BlockSpec((_A, _BT), lambda i: (0, i)),
            pl.BlockSpec((_A, _BT), lambda i: (0, i)),
            pl.BlockSpec((_BT,), lambda i: (i,)),
            pl.BlockSpec((_S, _BT), lambda i: (0, i)),
        ],
        out_shape=[
            jax.ShapeDtypeStruct((_A, _B), jnp.float32),
            jax.ShapeDtypeStruct((_A, _B), jnp.float32),
            jax.ShapeDtypeStruct((_B,), jnp.float32),
            jax.ShapeDtypeStruct((_S, _B), jnp.float32),
        ],
        compiler_params=pltpu.CompilerParams(
            dimension_semantics=("arbitrary",),
        ),
    )(state, w_rows, b_col)

    return (mean_t.T, std_t.T, value, onehot_t.T)


# speedup vs baseline: 5.8817x; 1.0728x over previous
import jax
import jax.numpy as jnp
from jax.experimental import pallas as pl
from jax.experimental.pallas import tpu as pltpu

_B, _D, _A, _S = 16384, 128, 64, 64
_BT = 8192
_NROW = 192
_NPAD = _NROW - (_A + _S + 1)


def _tc_body(state_ref, w_ref, b_ref, mean_ref, std_ref, value_ref, onehot_ref):
    x = state_ref[...]
    w = w_ref[...]
    y = jax.lax.dot_general(w, x, (((1,), (1,)), ((), ())),
                            preferred_element_type=jnp.float32)
    y = y + b_ref[...]
    mean_ref[...] = y[:_A, :]
    std_ref[...] = jnp.zeros((_A, _BT), jnp.float32)
    logits = y[_A:_A + _S, :]
    m = jnp.max(logits, axis=0, keepdims=True)
    iota = jax.lax.broadcasted_iota(jnp.int32, (_S, _BT), 0)
    first = jnp.min(jnp.where(logits == m, iota, _S), axis=0, keepdims=True)
    onehot_ref[...] = (iota == first).astype(jnp.float32)
    value_ref[...] = y[_A + _S, :]


def kernel(state, W_skill, b_skill, W_action, b_action, W_value, b_value):
    w_rows = jnp.concatenate(
        [W_action, W_skill, W_value, jnp.zeros((_NPAD, _D), jnp.float32)],
        axis=0)
    b_col = jnp.concatenate(
        [b_action, b_skill, b_value,
         jnp.zeros((_NPAD,), jnp.float32)]).reshape(_NROW, 1)

    mean_t, std_t, value, onehot_t = pl.pallas_call(
        _tc_body,
        grid=(_B // _BT,),
        in_specs=[
            pl.BlockSpec((_BT, _D), lambda i: (i, 0)),
            pl.BlockSpec((_NROW, _D), lambda i: (0, 0)),
            pl.BlockSpec((_NROW, 1), lambda i: (0, 0)),
        ],
        out_specs=[
            pl.BlockSpec((_A, _BT), lambda i: (0, i)),
            pl.BlockSpec((_A, _BT), lambda i: (0, i)),
            pl.BlockSpec((_BT,), lambda i: (i,)),
            pl.BlockSpec((_S, _BT), lambda i: (0, i)),
        ],
        out_shape=[
            jax.ShapeDtypeStruct((_A, _B), jnp.float32),
            jax.ShapeDtypeStruct((_A, _B), jnp.float32),
            jax.ShapeDtypeStruct((_B,), jnp.float32),
            jax.ShapeDtypeStruct((_S, _B), jnp.float32),
        ],
        compiler_params=pltpu.CompilerParams(
            dimension_semantics=("arbitrary",),
        ),
    )(state, w_rows, b_col)

    return (mean_t.T, std_t.T, value, onehot_t.T)
